# Initial kernel scaffold; baseline (speedup 1.0000x reference)
import jax
import jax.numpy as jnp
from jax.experimental import pallas as pl


def kernel(query_points, support_points, features, weight, offset_weight, offset_bias, kernel_points, bn_gamma, bn_beta, neighbors_indices):
    raise NotImplementedError("write your pallas kernel here")


# speedup vs baseline: 2.1834x; 2.1834x over previous
import functools

import jax
import jax.numpy as jnp
from jax import lax
from jax.experimental import pallas as pl
from jax.experimental.pallas import tpu as pltpu
from jax.experimental.pallas import tpu_sc as plsc

N_POINTS = 10000
N_NEIGH = 32
IN_FDIM = 128
OUT_FDIM = 128
NUM_KP = 15
KPAD = 16
RADIUS = 2.5
CFG_KP_EXTENT = 1.2
DENSITY = 5.0
EXTENT = CFG_KP_EXTENT * RADIUS / DENSITY

HIGH = None


_SC_CHUNK = 80
_NW = 32


def _sc_gather_body(idx_hbm, feat_hbm, spx_hbm, spy_hbm, spz_hbm,
                    fg_out, sx_out, sy_out, sz_out,
                    idx_v, fv, spx_v, spy_v, spz_v, sx_v, sy_v, sz_v, sem1):
    nc = 2
    wid = lax.axis_index("s") * nc + lax.axis_index("c")
    rows_per_w = (N_POINTS * N_NEIGH) // _NW
    nch = rows_per_w // _SC_CHUNK

    pltpu.sync_copy(spx_hbm, spx_v)
    pltpu.sync_copy(spy_hbm, spy_v)
    pltpu.sync_copy(spz_hbm, spz_v)

    def body(i, carry):
        base = wid * rows_per_w + i * _SC_CHUNK
        pltpu.sync_copy(idx_hbm.at[pl.ds(base, _SC_CHUNK)], idx_v)
        cp1 = pltpu.async_copy(feat_hbm.at[idx_v], fv, sem1)
        for j in range(_SC_CHUNK // 16):
            iv = idx_v[pl.ds(j * 16, 16)]
            sx_v[pl.ds(j * 16, 16)] = plsc.load_gather(spx_v, [iv])
            sy_v[pl.ds(j * 16, 16)] = plsc.load_gather(spy_v, [iv])
            sz_v[pl.ds(j * 16, 16)] = plsc.load_gather(spz_v, [iv])
        cp1.wait()
        pltpu.sync_copy(fv, fg_out.at[pl.ds(base, _SC_CHUNK)])
        pltpu.sync_copy(sx_v, sx_out.at[pl.ds(base, _SC_CHUNK)])
        pltpu.sync_copy(sy_v, sy_out.at[pl.ds(base, _SC_CHUNK)])
        pltpu.sync_copy(sz_v, sz_out.at[pl.ds(base, _SC_CHUNK)])
        return carry

    lax.fori_loop(0, nch, body, 0)


def _sc_gather(idx_flat, features, spx, spy, spz):
    total = N_POINTS * N_NEIGH
    mesh = plsc.VectorSubcoreMesh(core_axis_name="c", subcore_axis_name="s")
    fn = functools.partial(
        pl.kernel,
        out_type=[
            jax.ShapeDtypeStruct((total, IN_FDIM), jnp.float32),
            jax.ShapeDtypeStruct((total,), jnp.float32),
            jax.ShapeDtypeStruct((total,), jnp.float32),
            jax.ShapeDtypeStruct((total,), jnp.float32),
        ],
        mesh=mesh,
        scratch_types=[
            pltpu.VMEM((_SC_CHUNK,), jnp.int32),
            pltpu.VMEM((_SC_CHUNK, IN_FDIM), jnp.float32),
            pltpu.VMEM((N_POINTS,), jnp.float32),
            pltpu.VMEM((N_POINTS,), jnp.float32),
            pltpu.VMEM((N_POINTS,), jnp.float32),
            pltpu.VMEM((_SC_CHUNK,), jnp.float32),
            pltpu.VMEM((_SC_CHUNK,), jnp.float32),
            pltpu.VMEM((_SC_CHUNK,), jnp.float32),
            pltpu.SemaphoreType.DMA,
        ],
        compiler_params=pltpu.CompilerParams(needs_layout_passes=False),
    )(_sc_gather_body)
    return fn(idx_flat, features, spx, spy, spz)


_PB = 200


def _tc_main_body(fg_ref, sx_ref, sy_ref, sz_ref, q_ref, kp_ref, w_ref,
                  ow_ref, ob_ref, x_ref, sums_ref):
    i = pl.program_id(0)
    fg = fg_ref[...]
    q = q_ref[...]
    kp = kp_ref[...]

    relx = sx_ref[...] - q[:, 0:1]
    rely = sy_ref[...] - q[:, 1:2]
    relz = sz_ref[...] - q[:, 2:3]

    dx = relx[:, :, None] - kp[0, :][None, None, :]
    dy = rely[:, :, None] - kp[1, :][None, None, :]
    dz = relz[:, :, None] - kp[2, :][None, None, :]
    d2 = dx * dx + dy * dy + dz * dz
    kmask = lax.broadcasted_iota(jnp.int32, (1, 1, KPAD), 2) < NUM_KP
    w1 = jnp.where(kmask, jnp.maximum(1.0 - jnp.sqrt(d2) / EXTENT, 0.0), 0.0)

    wf1 = lax.dot_general(w1, fg, (((1,), (1,)), ((0,), (0,))),
                          precision=HIGH)

    of = ob_ref[...] * jnp.ones((_PB, 1), jnp.float32)
    for k in range(NUM_KP):
        of = of + lax.dot_general(wf1[:, k, :], ow_ref[k],
                                  (((1,), (0,)), ((), ())), precision=HIGH)
    of = of * EXTENT
    off3 = of.reshape(_PB, KPAD, KPAD)
    dkx = kp[0, :][None, :] + off3[:, :, 0]
    dky = kp[1, :][None, :] + off3[:, :, 1]
    dkz = kp[2, :][None, :] + off3[:, :, 2]

    dx2 = relx[:, :, None] - dkx[:, None, :]
    dy2 = rely[:, :, None] - dky[:, None, :]
    dz2 = relz[:, :, None] - dkz[:, None, :]
    d2b = dx2 * dx2 + dy2 * dy2 + dz2 * dz2
    w2 = jnp.where(kmask, jnp.maximum(1.0 - jnp.sqrt(d2b) / EXTENT, 0.0), 0.0)

    wf2 = lax.dot_general(w2, fg, (((1,), (1,)), ((0,), (0,))),
                          precision=HIGH)
    x = jnp.zeros((_PB, OUT_FDIM), jnp.float32)
    for k in range(NUM_KP):
        x = x + lax.dot_general(wf2[:, k, :], w_ref[k],
                                (((1,), (0,)), ((), ())), precision=HIGH)

    x_ref[...] = x

    @pl.when(i == 0)
    def _():
        sums_ref[...] = jnp.zeros_like(sums_ref)

    sums_ref[0:1, :] = sums_ref[0:1, :] + jnp.sum(x, axis=0, keepdims=True)
    sums_ref[1:2, :] = sums_ref[1:2, :] + jnp.sum(x * x, axis=0, keepdims=True)


def _tc_main(fg3, sx2, sy2, sz2, q4, kp3, w_pad, ow_pad, ob_pad):
    nb = N_POINTS // _PB
    return pl.pallas_call(
        _tc_main_body,
        grid=(nb,),
        in_specs=[
            pl.BlockSpec((_PB, N_NEIGH, IN_FDIM), lambda i: (i, 0, 0)),
            pl.BlockSpec((_PB, N_NEIGH), lambda i: (i, 0)),
            pl.BlockSpec((_PB, N_NEIGH), lambda i: (i, 0)),
            pl.BlockSpec((_PB, N_NEIGH), lambda i: (i, 0)),
            pl.BlockSpec((_PB, 4), lambda i: (i, 0)),
            pl.BlockSpec((3, KPAD), lambda i: (0, 0)),
            pl.BlockSpec((KPAD, IN_FDIM, OUT_FDIM), lambda i: (0, 0, 0)),
            pl.BlockSpec((KPAD, IN_FDIM, KPAD * KPAD), lambda i: (0, 0, 0)),
            pl.BlockSpec((1, KPAD * KPAD), lambda i: (0, 0)),
        ],
        out_specs=[
            pl.BlockSpec((_PB, OUT_FDIM), lambda i: (i, 0)),
            pl.BlockSpec((8, 128), lambda i: (0, 0)),
        ],
        out_shape=[
            jax.ShapeDtypeStruct((N_POINTS, OUT_FDIM), jnp.float32),
            jax.ShapeDtypeStruct((8, 128), jnp.float32),
        ],
    )(fg3, sx2, sy2, sz2, q4, kp3, w_pad, ow_pad, ob_pad)


_PB2 = 1000


def _tc_bn_body(x_ref, sums_ref, g_ref, b_ref, o_ref):
    n = jnp.float32(N_POINTS)
    mean = sums_ref[0:1, :] / n
    var = sums_ref[1:2, :] / n - mean * mean
    x = x_ref[...]
    y = (x - mean) / jnp.sqrt(var + 1e-6) * g_ref[...] + b_ref[...]
    o_ref[...] = jnp.where(y >= 0.0, y, 0.1 * y)


def _tc_bn(x, sums, gamma2, beta2):
    nb = N_POINTS // _PB2
    return pl.pallas_call(
        _tc_bn_body,
        grid=(nb,),
        in_specs=[
            pl.BlockSpec((_PB2, OUT_FDIM), lambda i: (i, 0)),
            pl.BlockSpec((8, 128), lambda i: (0, 0)),
            pl.BlockSpec((1, OUT_FDIM), lambda i: (0, 0)),
            pl.BlockSpec((1, OUT_FDIM), lambda i: (0, 0)),
        ],
        out_specs=pl.BlockSpec((_PB2, OUT_FDIM), lambda i: (i, 0)),
        out_shape=jax.ShapeDtypeStruct((N_POINTS, OUT_FDIM), jnp.float32),
    )(x, sums, gamma2, beta2)


def kernel(query_points, support_points, features, weight, offset_weight,
           offset_bias, kernel_points, bn_gamma, bn_beta, neighbors_indices):
    idx_flat = neighbors_indices.reshape(-1).astype(jnp.int32)
    spx = support_points[:, 0]
    spy = support_points[:, 1]
    spz = support_points[:, 2]
    q4 = jnp.pad(query_points, ((0, 0), (0, 1)))
    kp3 = jnp.pad(kernel_points, ((0, KPAD - NUM_KP), (0, 0))).T
    w_pad = jnp.pad(weight, ((0, KPAD - NUM_KP), (0, 0), (0, 0)))
    ow = offset_weight.reshape(NUM_KP, IN_FDIM, NUM_KP, 3)
    ow_pad = jnp.pad(ow, ((0, KPAD - NUM_KP), (0, 0), (0, KPAD - NUM_KP),
                          (0, KPAD - 3)))
    ow_pad = ow_pad.reshape(KPAD, IN_FDIM, KPAD * KPAD)
    ob = offset_bias.reshape(NUM_KP, 3)
    ob_pad = jnp.pad(ob, ((0, KPAD - NUM_KP), (0, KPAD - 3)))
    ob_pad = ob_pad.reshape(1, KPAD * KPAD)
    gamma2 = bn_gamma.reshape(1, OUT_FDIM)
    beta2 = bn_beta.reshape(1, OUT_FDIM)

    fg, sx, sy, sz = _sc_gather(idx_flat, features, spx, spy, spz)
    fg3 = fg.reshape(N_POINTS, N_NEIGH, IN_FDIM)
    sx2 = sx.reshape(N_POINTS, N_NEIGH)
    sy2 = sy.reshape(N_POINTS, N_NEIGH)
    sz2 = sz.reshape(N_POINTS, N_NEIGH)

    x, sums = _tc_main(fg3, sx2, sy2, sz2, q4, kp3, w_pad, ow_pad, ob_pad)
    return _tc_bn(x, sums, gamma2, beta2)


# speedup vs baseline: 3.0935x; 1.4168x over previous
import functools

import jax
import jax.numpy as jnp
from jax import lax
from jax.experimental import pallas as pl
from jax.experimental.pallas import tpu as pltpu
from jax.experimental.pallas import tpu_sc as plsc

N_POINTS = 10000
N_NEIGH = 32
IN_FDIM = 128
OUT_FDIM = 128
NUM_KP = 15
KPAD = 16
RADIUS = 2.5
CFG_KP_EXTENT = 1.2
DENSITY = 5.0
EXTENT = CFG_KP_EXTENT * RADIUS / DENSITY

HIGH = None


_SC_CHUNK = 80
_NW = 32


def _sc_gather_body(idx_hbm, feat_hbm, spx_hbm, spy_hbm, spz_hbm,
                    fg_out, sx_out, sy_out, sz_out,
                    idx_v, fv, spx_v, spy_v, spz_v, sx_v, sy_v, sz_v, sem1):
    nc = 2
    wid = lax.axis_index("s") * nc + lax.axis_index("c")
    rows_per_w = (N_POINTS * N_NEIGH) // _NW
    nch = rows_per_w // _SC_CHUNK

    pltpu.sync_copy(spx_hbm, spx_v)
    pltpu.sync_copy(spy_hbm, spy_v)
    pltpu.sync_copy(spz_hbm, spz_v)

    def body(i, carry):
        base = wid * rows_per_w + i * _SC_CHUNK
        pltpu.sync_copy(idx_hbm.at[pl.ds(base, _SC_CHUNK)], idx_v)
        cp1 = pltpu.async_copy(feat_hbm.at[idx_v], fv, sem1)
        for j in range(_SC_CHUNK // 16):
            iv = idx_v[pl.ds(j * 16, 16)]
            sx_v[pl.ds(j * 16, 16)] = plsc.load_gather(spx_v, [iv])
            sy_v[pl.ds(j * 16, 16)] = plsc.load_gather(spy_v, [iv])
            sz_v[pl.ds(j * 16, 16)] = plsc.load_gather(spz_v, [iv])
        cp1.wait()
        pltpu.sync_copy(fv, fg_out.at[pl.ds(base, _SC_CHUNK)])
        pltpu.sync_copy(sx_v, sx_out.at[pl.ds(base, _SC_CHUNK)])
        pltpu.sync_copy(sy_v, sy_out.at[pl.ds(base, _SC_CHUNK)])
        pltpu.sync_copy(sz_v, sz_out.at[pl.ds(base, _SC_CHUNK)])
        return carry

    lax.fori_loop(0, nch, body, 0)


def _sc_gather(idx_flat, features, spx, spy, spz):
    total = N_POINTS * N_NEIGH
    mesh = plsc.VectorSubcoreMesh(core_axis_name="c", subcore_axis_name="s")
    fn = functools.partial(
        pl.kernel,
        out_type=[
            jax.ShapeDtypeStruct((total, IN_FDIM), jnp.float32),
            jax.ShapeDtypeStruct((total,), jnp.float32),
            jax.ShapeDtypeStruct((total,), jnp.float32),
            jax.ShapeDtypeStruct((total,), jnp.float32),
        ],
        mesh=mesh,
        scratch_types=[
            pltpu.VMEM((_SC_CHUNK,), jnp.int32),
            pltpu.VMEM((_SC_CHUNK, IN_FDIM), jnp.float32),
            pltpu.VMEM((N_POINTS,), jnp.float32),
            pltpu.VMEM((N_POINTS,), jnp.float32),
            pltpu.VMEM((N_POINTS,), jnp.float32),
            pltpu.VMEM((_SC_CHUNK,), jnp.float32),
            pltpu.VMEM((_SC_CHUNK,), jnp.float32),
            pltpu.VMEM((_SC_CHUNK,), jnp.float32),
            pltpu.SemaphoreType.DMA,
        ],
        compiler_params=pltpu.CompilerParams(needs_layout_passes=False),
    )(_sc_gather_body)
    return fn(idx_flat, features, spx, spy, spz)


_PB = 200


def _tc_main_body(fg_ref, sx_ref, sy_ref, sz_ref, q_ref, kps_ref, kpl_ref,
                  w_ref, ow_ref, ob_ref, x_ref, sums_ref):
    i = pl.program_id(0)
    fg = fg_ref[...]
    q = q_ref[...]
    kps = kps_ref[...]
    kpl = kpl_ref[...]

    relx = sx_ref[...] - q[:, 0:1]
    rely = sy_ref[...] - q[:, 1:2]
    relz = sz_ref[...] - q[:, 2:3]

    dx = relx[:, None, :] - kps[:, 0:1][None, :, :]
    dy = rely[:, None, :] - kps[:, 1:2][None, :, :]
    dz = relz[:, None, :] - kps[:, 2:3][None, :, :]
    d2 = dx * dx + dy * dy + dz * dz
    kmask = lax.broadcasted_iota(jnp.int32, (1, KPAD, 1), 1) < NUM_KP
    w1 = jnp.where(kmask, jnp.maximum(1.0 - jnp.sqrt(d2) / EXTENT, 0.0), 0.0)

    wf1 = lax.dot_general(w1, fg, (((2,), (1,)), ((0,), (0,))),
                          precision=HIGH)

    of = ob_ref[...] * jnp.ones((_PB, 1), jnp.float32)
    for k in range(NUM_KP):
        of = of + lax.dot_general(wf1[:, k, :], ow_ref[k],
                                  (((1,), (0,)), ((), ())), precision=HIGH)
    of = of * EXTENT
    dkx = kpl[0:1, :] + of[:, 0:16]
    dky = kpl[1:2, :] + of[:, 16:32]
    dkz = kpl[2:3, :] + of[:, 32:48]

    dx2 = relx[:, None, :] - dkx[:, :, None]
    dy2 = rely[:, None, :] - dky[:, :, None]
    dz2 = relz[:, None, :] - dkz[:, :, None]
    d2b = dx2 * dx2 + dy2 * dy2 + dz2 * dz2
    w2 = jnp.where(kmask, jnp.maximum(1.0 - jnp.sqrt(d2b) / EXTENT, 0.0), 0.0)

    wf2 = lax.dot_general(w2, fg, (((2,), (1,)), ((0,), (0,))),
                          precision=HIGH)
    x = jnp.zeros((_PB, OUT_FDIM), jnp.float32)
    for k in range(NUM_KP):
        x = x + lax.dot_general(wf2[:, k, :], w_ref[k],
                                (((1,), (0,)), ((), ())), precision=HIGH)

    x_ref[...] = x

    @pl.when(i == 0)
    def _():
        sums_ref[...] = jnp.zeros_like(sums_ref)

    sums_ref[0:1, :] = sums_ref[0:1, :] + jnp.sum(x, axis=0, keepdims=True)
    sums_ref[1:2, :] = sums_ref[1:2, :] + jnp.sum(x * x, axis=0, keepdims=True)


def _tc_main(fg3, sx2, sy2, sz2, q4, kp_s, kp_l, w_pad, ow_pad, ob_pad):
    nb = N_POINTS // _PB
    return pl.pallas_call(
        _tc_main_body,
        grid=(nb,),
        in_specs=[
            pl.BlockSpec((_PB, N_NEIGH, IN_FDIM), lambda i: (i, 0, 0)),
            pl.BlockSpec((_PB, N_NEIGH), lambda i: (i, 0)),
            pl.BlockSpec((_PB, N_NEIGH), lambda i: (i, 0)),
            pl.BlockSpec((_PB, N_NEIGH), lambda i: (i, 0)),
            pl.BlockSpec((_PB, 4), lambda i: (i, 0)),
            pl.BlockSpec((KPAD, 4), lambda i: (0, 0)),
            pl.BlockSpec((3, KPAD), lambda i: (0, 0)),
            pl.BlockSpec((KPAD, IN_FDIM, OUT_FDIM), lambda i: (0, 0, 0)),
            pl.BlockSpec((KPAD, IN_FDIM, 128), lambda i: (0, 0, 0)),
            pl.BlockSpec((1, 128), lambda i: (0, 0)),
        ],
        out_specs=[
            pl.BlockSpec((_PB, OUT_FDIM), lambda i: (i, 0)),
            pl.BlockSpec((8, 128), lambda i: (0, 0)),
        ],
        out_shape=[
            jax.ShapeDtypeStruct((N_POINTS, OUT_FDIM), jnp.float32),
            jax.ShapeDtypeStruct((8, 128), jnp.float32),
        ],
    )(fg3, sx2, sy2, sz2, q4, kp_s, kp_l, w_pad, ow_pad, ob_pad)


_PB2 = 1000


def _tc_bn_body(x_ref, sums_ref, g_ref, b_ref, o_ref):
    n = jnp.float32(N_POINTS)
    mean = sums_ref[0:1, :] / n
    var = sums_ref[1:2, :] / n - mean * mean
    x = x_ref[...]
    y = (x - mean) / jnp.sqrt(var + 1e-6) * g_ref[...] + b_ref[...]
    o_ref[...] = jnp.where(y >= 0.0, y, 0.1 * y)


def _tc_bn(x, sums, gamma2, beta2):
    nb = N_POINTS // _PB2
    return pl.pallas_call(
        _tc_bn_body,
        grid=(nb,),
        in_specs=[
            pl.BlockSpec((_PB2, OUT_FDIM), lambda i: (i, 0)),
            pl.BlockSpec((8, 128), lambda i: (0, 0)),
            pl.BlockSpec((1, OUT_FDIM), lambda i: (0, 0)),
            pl.BlockSpec((1, OUT_FDIM), lambda i: (0, 0)),
        ],
        out_specs=pl.BlockSpec((_PB2, OUT_FDIM), lambda i: (i, 0)),
        out_shape=jax.ShapeDtypeStruct((N_POINTS, OUT_FDIM), jnp.float32),
    )(x, sums, gamma2, beta2)


def kernel(query_points, support_points, features, weight, offset_weight,
           offset_bias, kernel_points, bn_gamma, bn_beta, neighbors_indices):
    idx_flat = neighbors_indices.reshape(-1).astype(jnp.int32)
    spx = support_points[:, 0]
    spy = support_points[:, 1]
    spz = support_points[:, 2]
    q4 = jnp.pad(query_points, ((0, 0), (0, 1)))
    kp_s = jnp.pad(kernel_points, ((0, KPAD - NUM_KP), (0, 1)))
    kp_l = jnp.pad(kernel_points, ((0, KPAD - NUM_KP), (0, 0))).T
    w_pad = jnp.pad(weight, ((0, KPAD - NUM_KP), (0, 0), (0, 0)))
    ow = offset_weight.reshape(NUM_KP, IN_FDIM, NUM_KP, 3)
    ow = jnp.transpose(ow, (0, 1, 3, 2))
    ow_pad = jnp.pad(ow, ((0, KPAD - NUM_KP), (0, 0), (0, 5),
                          (0, KPAD - NUM_KP)))
    ow_pad = ow_pad.reshape(KPAD, IN_FDIM, 128)
    ob = offset_bias.reshape(NUM_KP, 3).T
    ob_pad = jnp.pad(ob, ((0, 5), (0, KPAD - NUM_KP)))
    ob_pad = ob_pad.reshape(1, 128)
    gamma2 = bn_gamma.reshape(1, OUT_FDIM)
    beta2 = bn_beta.reshape(1, OUT_FDIM)

    fg, sx, sy, sz = _sc_gather(idx_flat, features, spx, spy, spz)
    fg3 = fg.reshape(N_POINTS, N_NEIGH, IN_FDIM)
    sx2 = sx.reshape(N_POINTS, N_NEIGH)
    sy2 = sy.reshape(N_POINTS, N_NEIGH)
    sz2 = sz.reshape(N_POINTS, N_NEIGH)

    x, sums = _tc_main(fg3, sx2, sy2, sz2, q4, kp_s, kp_l, w_pad, ow_pad,
                       ob_pad)
    return _tc_bn(x, sums, gamma2, beta2)


# speedup vs baseline: 3.6691x; 1.1861x over previous
import functools

import jax
import jax.numpy as jnp
from jax import lax
from jax.experimental import pallas as pl
from jax.experimental.pallas import tpu as pltpu
from jax.experimental.pallas import tpu_sc as plsc

N_POINTS = 10000
N_NEIGH = 32
IN_FDIM = 128
OUT_FDIM = 128
NUM_KP = 15
KPAD = 16
RADIUS = 2.5
CFG_KP_EXTENT = 1.2
DENSITY = 5.0
EXTENT = CFG_KP_EXTENT * RADIUS / DENSITY

HIGH = None


_SC_CHUNK = 40
_NW = 32


def _sc_gather_body(idx_hbm, feat_hbm, spx_hbm, spy_hbm, spz_hbm,
                    fg_out, sx_out, sy_out, sz_out,
                    idx_v, fv0, fv1, spx_v, spy_v, spz_v,
                    sx0, sy0, sz0, sx1, sy1, sz1,
                    semg0, semg1, semw0, semw1):
    nc = 2
    wid = lax.axis_index("s") * nc + lax.axis_index("c")
    rows_per_w = (N_POINTS * N_NEIGH) // _NW
    npair = rows_per_w // (2 * _SC_CHUNK)

    pltpu.sync_copy(idx_hbm.at[pl.ds(wid * rows_per_w, rows_per_w)],
                    idx_v.at[pl.ds(0, rows_per_w)])
    pltpu.sync_copy(spx_hbm, spx_v)
    pltpu.sync_copy(spy_hbm, spy_v)
    pltpu.sync_copy(spz_hbm, spz_v)

    bufs = ((fv0, sx0, sy0, sz0, semg0, semw0),
            (fv1, sx1, sy1, sz1, semg1, semw1))

    def wait_writes(j, base, b):
        fv, sx, sy, sz, _, semw = bufs[b]
        @pl.when(j > 0)
        def _():
            pltpu.make_async_copy(fv, fg_out.at[pl.ds(base, _SC_CHUNK)],
                                  semw).wait()
            pltpu.make_async_copy(sx.at[pl.ds(0, _SC_CHUNK)],
                                  sx_out.at[pl.ds(base, _SC_CHUNK)],
                                  semw).wait()
            pltpu.make_async_copy(sy.at[pl.ds(0, _SC_CHUNK)],
                                  sy_out.at[pl.ds(base, _SC_CHUNK)],
                                  semw).wait()
            pltpu.make_async_copy(sz.at[pl.ds(0, _SC_CHUNK)],
                                  sz_out.at[pl.ds(base, _SC_CHUNK)],
                                  semw).wait()

    def body(j, carry):
        base0 = wid * rows_per_w + 2 * j * _SC_CHUNK
        base1 = base0 + _SC_CHUNK
        off0 = 2 * j * _SC_CHUNK
        off1 = off0 + _SC_CHUNK
        wait_writes(j, base0, 0)
        wait_writes(j, base1, 1)
        g0 = pltpu.async_copy(feat_hbm.at[idx_v.at[pl.ds(off0, _SC_CHUNK)]],
                              fv0, semg0)
        g1 = pltpu.async_copy(feat_hbm.at[idx_v.at[pl.ds(off1, _SC_CHUNK)]],
                              fv1, semg1)
        for (off, (fv, sx, sy, sz, semg, semw)) in ((off0, bufs[0]),
                                                    (off1, bufs[1])):
            for t in range((_SC_CHUNK + 15) // 16):
                iv = idx_v[pl.ds(off + t * 16, 16)]
                iv = jnp.minimum(iv, N_POINTS - 1)
                sx[pl.ds(t * 16, 16)] = plsc.load_gather(spx_v, [iv])
                sy[pl.ds(t * 16, 16)] = plsc.load_gather(spy_v, [iv])
                sz[pl.ds(t * 16, 16)] = plsc.load_gather(spz_v, [iv])
        g0.wait()
        pltpu.async_copy(fv0, fg_out.at[pl.ds(base0, _SC_CHUNK)], semw0)
        pltpu.async_copy(sx0.at[pl.ds(0, _SC_CHUNK)],
                         sx_out.at[pl.ds(base0, _SC_CHUNK)], semw0)
        pltpu.async_copy(sy0.at[pl.ds(0, _SC_CHUNK)],
                         sy_out.at[pl.ds(base0, _SC_CHUNK)], semw0)
        pltpu.async_copy(sz0.at[pl.ds(0, _SC_CHUNK)],
                         sz_out.at[pl.ds(base0, _SC_CHUNK)], semw0)
        g1.wait()
        pltpu.async_copy(fv1, fg_out.at[pl.ds(base1, _SC_CHUNK)], semw1)
        pltpu.async_copy(sx1.at[pl.ds(0, _SC_CHUNK)],
                         sx_out.at[pl.ds(base1, _SC_CHUNK)], semw1)
        pltpu.async_copy(sy1.at[pl.ds(0, _SC_CHUNK)],
                         sy_out.at[pl.ds(base1, _SC_CHUNK)], semw1)
        pltpu.async_copy(sz1.at[pl.ds(0, _SC_CHUNK)],
                         sz_out.at[pl.ds(base1, _SC_CHUNK)], semw1)
        return carry

    lax.fori_loop(0, npair, body, 0)
    tail0 = wid * rows_per_w
    wait_writes(jnp.int32(1), tail0, 0)
    wait_writes(jnp.int32(1), tail0 + _SC_CHUNK, 1)


def _sc_gather(idx_flat, features, spx, spy, spz):
    total = N_POINTS * N_NEIGH
    rows_per_w = total // _NW
    mesh = plsc.VectorSubcoreMesh(core_axis_name="c", subcore_axis_name="s")
    cb = 16 * ((_SC_CHUNK + 15) // 16)
    coordbuf = [pltpu.VMEM((cb,), jnp.float32) for _ in range(6)]
    fn = functools.partial(
        pl.kernel,
        out_type=[
            jax.ShapeDtypeStruct((total, IN_FDIM), jnp.float32),
            jax.ShapeDtypeStruct((total,), jnp.float32),
            jax.ShapeDtypeStruct((total,), jnp.float32),
            jax.ShapeDtypeStruct((total,), jnp.float32),
        ],
        mesh=mesh,
        scratch_types=[
            pltpu.VMEM((rows_per_w + 16,), jnp.int32),
            pltpu.VMEM((_SC_CHUNK, IN_FDIM), jnp.float32),
            pltpu.VMEM((_SC_CHUNK, IN_FDIM), jnp.float32),
            pltpu.VMEM((N_POINTS,), jnp.float32),
            pltpu.VMEM((N_POINTS,), jnp.float32),
            pltpu.VMEM((N_POINTS,), jnp.float32),
        ] + coordbuf + [
            pltpu.SemaphoreType.DMA,
            pltpu.SemaphoreType.DMA,
            pltpu.SemaphoreType.DMA,
            pltpu.SemaphoreType.DMA,
        ],
        compiler_params=pltpu.CompilerParams(needs_layout_passes=False),
    )(_sc_gather_body)
    return fn(idx_flat, features, spx, spy, spz)


_PB = 200


def _tc_main_body(fg_ref, sx_ref, sy_ref, sz_ref, q_ref, kps_ref, kpl_ref,
                  w_ref, ow_ref, ob_ref, x_ref, sums_ref):
    i = pl.program_id(0)
    fg = fg_ref[...]
    q = q_ref[...]
    kps = kps_ref[...]
    kpl = kpl_ref[...]

    relx = sx_ref[...] - q[:, 0:1]
    rely = sy_ref[...] - q[:, 1:2]
    relz = sz_ref[...] - q[:, 2:3]

    dx = relx[:, None, :] - kps[:, 0:1][None, :, :]
    dy = rely[:, None, :] - kps[:, 1:2][None, :, :]
    dz = relz[:, None, :] - kps[:, 2:3][None, :, :]
    d2 = dx * dx + dy * dy + dz * dz
    kmask = lax.broadcasted_iota(jnp.int32, (1, KPAD, 1), 1) < NUM_KP
    w1 = jnp.where(kmask, jnp.maximum(1.0 - jnp.sqrt(d2) / EXTENT, 0.0), 0.0)

    wf1 = lax.dot_general(w1, fg, (((2,), (1,)), ((0,), (0,))),
                          precision=HIGH)

    of = ob_ref[...] * jnp.ones((_PB, 1), jnp.float32)
    for k in range(NUM_KP):
        of = of + lax.dot_general(wf1[:, k, :], ow_ref[k],
                                  (((1,), (0,)), ((), ())), precision=HIGH)
    of = of * EXTENT
    dkx = kpl[0:1, :] + of[:, 0:16]
    dky = kpl[1:2, :] + of[:, 16:32]
    dkz = kpl[2:3, :] + of[:, 32:48]

    dx2 = relx[:, None, :] - dkx[:, :, None]
    dy2 = rely[:, None, :] - dky[:, :, None]
    dz2 = relz[:, None, :] - dkz[:, :, None]
    d2b = dx2 * dx2 + dy2 * dy2 + dz2 * dz2
    w2 = jnp.where(kmask, jnp.maximum(1.0 - jnp.sqrt(d2b) / EXTENT, 0.0), 0.0)

    wf2 = lax.dot_general(w2, fg, (((2,), (1,)), ((0,), (0,))),
                          precision=HIGH)
    x = jnp.zeros((_PB, OUT_FDIM), jnp.float32)
    for k in range(NUM_KP):
        x = x + lax.dot_general(wf2[:, k, :], w_ref[k],
                                (((1,), (0,)), ((), ())), precision=HIGH)

    x_ref[...] = x

    @pl.when(i == 0)
    def _():
        sums_ref[...] = jnp.zeros_like(sums_ref)

    sums_ref[0:1, :] = sums_ref[0:1, :] + jnp.sum(x, axis=0, keepdims=True)
    sums_ref[1:2, :] = sums_ref[1:2, :] + jnp.sum(x * x, axis=0, keepdims=True)


def _tc_main(fg3, sx2, sy2, sz2, q4, kp_s, kp_l, w_pad, ow_pad, ob_pad):
    nb = N_POINTS // _PB
    return pl.pallas_call(
        _tc_main_body,
        grid=(nb,),
        in_specs=[
            pl.BlockSpec((_PB, N_NEIGH, IN_FDIM), lambda i: (i, 0, 0)),
            pl.BlockSpec((_PB, N_NEIGH), lambda i: (i, 0)),
            pl.BlockSpec((_PB, N_NEIGH), lambda i: (i, 0)),
            pl.BlockSpec((_PB, N_NEIGH), lambda i: (i, 0)),
            pl.BlockSpec((_PB, 4), lambda i: (i, 0)),
            pl.BlockSpec((KPAD, 4), lambda i: (0, 0)),
            pl.BlockSpec((3, KPAD), lambda i: (0, 0)),
            pl.BlockSpec((KPAD, IN_FDIM, OUT_FDIM), lambda i: (0, 0, 0)),
            pl.BlockSpec((KPAD, IN_FDIM, 128), lambda i: (0, 0, 0)),
            pl.BlockSpec((1, 128), lambda i: (0, 0)),
        ],
        out_specs=[
            pl.BlockSpec((_PB, OUT_FDIM), lambda i: (i, 0)),
            pl.BlockSpec((8, 128), lambda i: (0, 0)),
        ],
        out_shape=[
            jax.ShapeDtypeStruct((N_POINTS, OUT_FDIM), jnp.float32),
            jax.ShapeDtypeStruct((8, 128), jnp.float32),
        ],
    )(fg3, sx2, sy2, sz2, q4, kp_s, kp_l, w_pad, ow_pad, ob_pad)


_PB2 = 1000


def _tc_bn_body(x_ref, sums_ref, g_ref, b_ref, o_ref):
    n = jnp.float32(N_POINTS)
    mean = sums_ref[0:1, :] / n
    var = sums_ref[1:2, :] / n - mean * mean
    x = x_ref[...]
    y = (x - mean) / jnp.sqrt(var + 1e-6) * g_ref[...] + b_ref[...]
    o_ref[...] = jnp.where(y >= 0.0, y, 0.1 * y)


def _tc_bn(x, sums, gamma2, beta2):
    nb = N_POINTS // _PB2
    return pl.pallas_call(
        _tc_bn_body,
        grid=(nb,),
        in_specs=[
            pl.BlockSpec((_PB2, OUT_FDIM), lambda i: (i, 0)),
            pl.BlockSpec((8, 128), lambda i: (0, 0)),
            pl.BlockSpec((1, OUT_FDIM), lambda i: (0, 0)),
            pl.BlockSpec((1, OUT_FDIM), lambda i: (0, 0)),
        ],
        out_specs=pl.BlockSpec((_PB2, OUT_FDIM), lambda i: (i, 0)),
        out_shape=jax.ShapeDtypeStruct((N_POINTS, OUT_FDIM), jnp.float32),
    )(x, sums, gamma2, beta2)


def kernel(query_points, support_points, features, weight, offset_weight,
           offset_bias, kernel_points, bn_gamma, bn_beta, neighbors_indices):
    idx_flat = neighbors_indices.reshape(-1).astype(jnp.int32)
    spx = support_points[:, 0]
    spy = support_points[:, 1]
    spz = support_points[:, 2]
    q4 = jnp.pad(query_points, ((0, 0), (0, 1)))
    kp_s = jnp.pad(kernel_points, ((0, KPAD - NUM_KP), (0, 1)))
    kp_l = jnp.pad(kernel_points, ((0, KPAD - NUM_KP), (0, 0))).T
    w_pad = jnp.pad(weight, ((0, KPAD - NUM_KP), (0, 0), (0, 0)))
    ow = offset_weight.reshape(NUM_KP, IN_FDIM, NUM_KP, 3)
    ow = jnp.transpose(ow, (0, 1, 3, 2))
    ow_pad = jnp.pad(ow, ((0, KPAD - NUM_KP), (0, 0), (0, 5),
                          (0, KPAD - NUM_KP)))
    ow_pad = ow_pad.reshape(KPAD, IN_FDIM, 128)
    ob = offset_bias.reshape(NUM_KP, 3).T
    ob_pad = jnp.pad(ob, ((0, 5), (0, KPAD - NUM_KP)))
    ob_pad = ob_pad.reshape(1, 128)
    gamma2 = bn_gamma.reshape(1, OUT_FDIM)
    beta2 = bn_beta.reshape(1, OUT_FDIM)

    fg, sx, sy, sz = _sc_gather(idx_flat, features, spx, spy, spz)
    fg3 = fg.reshape(N_POINTS, N_NEIGH, IN_FDIM)
    sx2 = sx.reshape(N_POINTS, N_NEIGH)
    sy2 = sy.reshape(N_POINTS, N_NEIGH)
    sz2 = sz.reshape(N_POINTS, N_NEIGH)

    x, sums = _tc_main(fg3, sx2, sy2, sz2, q4, kp_s, kp_l, w_pad, ow_pad,
                       ob_pad)
    return _tc_bn(x, sums, gamma2, beta2)
